# baseline (speedup 1.0000x reference)
import jax
import jax.numpy as jnp
from jax import lax
from jax.experimental import pallas as pl
from jax.experimental.pallas import tpu as pltpu
from jax.experimental.pallas import tpu_sc as plsc

_ATOM_WEIGHTS = [1.008, 12.011, 14.007, 15.999, 18.998, 30.974, 32.06,
                 35.453, 79.904, 126.904, 10.811, 28.086, 78.971, 22.99,
                 39.098, 6.941]
_MAX_WEIGHT = 126.904
_WNORM = [w / _MAX_WEIGHT for w in _ATOM_WEIGHTS]

_B = 16384
_A = 200
_C = 16
_NW = 32
_MOLS_PER_W = _B // _NW
_MOL_WORDS = _A * _C
_P = 8
_CHUNKS = _MOLS_PER_W // _P
_CHUNK_WORDS = _P * _MOL_WORDS
_PAIRS = _P // 2
_GROUPS = 2 * _A // _C
_PAIR_WORDS = 2 * _MOL_WORDS


def _argmax_weight(vals, weights):
  items = list(zip(vals, weights))
  while len(items) > 1:
    nxt = []
    for i in range(0, len(items), 2):
      v1, w1 = items[i]
      v2, w2 = items[i + 1]
      gt = v2 > v1
      nxt.append((jnp.where(gt, v2, v1), jnp.where(gt, w2, w1)))
    items = nxt
  return items[0][1]


def _tec_body(x_hbm, out_hbm, buf, sums, outv):
  wid = lax.axis_index("s") * 2 + lax.axis_index("c")
  mol0 = wid * _MOLS_PER_W

  lane = lax.iota(jnp.int32, 16)
  lane16 = lane * _C
  lo_half = lane < 8
  zero = jnp.zeros((16,), jnp.float32)
  wsplats = [jnp.full((16,), w, jnp.float32) for w in _WNORM]

  def pair_body(base, acc_a, acc_b):
    for g in range(_GROUPS):
      gb = base + g * (_C * _C)
      vals = [plsc.load_gather(buf, [gb + c + lane16]) for c in range(_C)]
      w = _argmax_weight(vals, wsplats)
      if g < _GROUPS // 2:
        acc_a = acc_a + w
      elif g == _GROUPS // 2:
        acc_a = acc_a + jnp.where(lo_half, w, zero)
        acc_b = acc_b + jnp.where(lo_half, zero, w)
      else:
        acc_b = acc_b + w
    return acc_a, acc_b

  def chunk_body(ci, carry):
    word0 = (mol0 + ci * _P) * _MOL_WORDS
    pltpu.sync_copy(x_hbm.at[pl.ds(word0, _CHUNK_WORDS)], buf)

    def pair_loop(p, c2):
      acc_a, acc_b = pair_body(p * _PAIR_WORDS, zero, zero)
      row = (ci * _P + 2 * p) * _C
      sums[pl.ds(row, _C)] = acc_a
      sums[pl.ds(row + _C, _C)] = acc_b
      return c2

    return lax.fori_loop(0, _PAIRS, pair_loop, carry)

  lax.fori_loop(0, _CHUNKS, chunk_body, 0)

  def red_body(mg, carry):
    base = mg * (_C * _C)
    tot = zero
    for j in range(_C):
      tot = tot + plsc.load_gather(sums, [base + j + lane16])
    outv[pl.ds(mg * _C, _C)] = tot
    return carry

  lax.fori_loop(0, _MOLS_PER_W // _C, red_body, 0)

  pltpu.sync_copy(outv, out_hbm.at[pl.ds(mol0, _MOLS_PER_W)])


_mesh = plsc.VectorSubcoreMesh(core_axis_name="c", subcore_axis_name="s")


@jax.jit
def _weight_feature(x_flat):
  return pl.kernel(
      _tec_body,
      out_type=jax.ShapeDtypeStruct((_B,), jnp.float32),
      mesh=_mesh,
      scratch_types=[
          pltpu.VMEM((_CHUNK_WORDS,), jnp.float32),
          pltpu.VMEM((_MOLS_PER_W * _C,), jnp.float32),
          pltpu.VMEM((_MOLS_PER_W,), jnp.float32),
      ],
      compiler_params=pltpu.CompilerParams(needs_layout_passes=False),
  )(x_flat)


def kernel(X):
  out = _weight_feature(X.reshape(-1))
  return out.reshape(_B, 1)
